# baseline (speedup 1.0000x reference)
import functools

import jax
import jax.numpy as jnp
from jax import lax
from jax.experimental import pallas as pl
from jax.experimental.pallas import tpu as pltpu
from jax.experimental.pallas import tpu_sc as plsc

NC = 2
NS = 16
IW = 128
WZ = 624


def _subcore_range(s, total_rows):
  q, r = divmod(total_rows, NS)
  start = s * q + jnp.minimum(s, r)
  cnt = q + jnp.where(s < r, 1, 0)
  return start, cnt


def _split_copy(src, dst, s, n_rows):
  tail = n_rows - NS * WZ
  pltpu.sync_copy(src.at[pl.ds(s * WZ, WZ)], dst.at[pl.ds(s * WZ, WZ)])
  if tail:
    @pl.when(s == 0)
    def _():
      pltpu.sync_copy(src.at[pl.ds(NS * WZ, tail)],
                      dst.at[pl.ds(NS * WZ, tail)])


def _sc_segsum(fetch_list, idx2_list, zeros_h, n_dst, n_pad, width):
  n_phases = len(fetch_list[0])
  wz_acc = n_pad // NS

  n_out = NC * n_phases
  out_type = (jax.ShapeDtypeStruct((n_dst, width), jnp.float32),) * n_out

  data_args = []
  layout = []
  for c in range(NC):
    ph_idx = []
    for kind, arr in fetch_list[c]:
      data_args.append(arr)
      ph_idx.append(len(data_args) - 1)
    src2, dst2 = idx2_list[c]
    s_i = None
    if src2 is not None:
      data_args.append(src2)
      s_i = len(data_args) - 1
    data_args.append(dst2)
    layout.append((ph_idx, s_i, len(data_args) - 1))
  data_args.append(zeros_h)
  z_i = len(data_args) - 1

  gather_mode = fetch_list[0][0][0] == "gather"
  n_didx = 3
  if gather_mode:
    rows_scratch = [pltpu.VMEM((IW, width), jnp.float32)] * 3
  else:
    rows_scratch = [pltpu.VMEM((IW, 16), jnp.float32)] * 2 + [
        pltpu.VMEM((IW, width), jnp.float32)]
  n_rows = len(rows_scratch)

  scratch = ([pltpu.VMEM((n_didx, IW), jnp.int32)]
             + [pltpu.VMEM((IW,), jnp.int32)] * 3
             + rows_scratch
             + [pltpu.SemaphoreType.DMA] * 3
             + [pltpu.VMEM_SHARED((n_pad, width), jnp.float32)])

  @functools.partial(
      pl.kernel,
      mesh=plsc.VectorSubcoreMesh(core_axis_name="c", subcore_axis_name="s"),
      out_type=out_type,
      scratch_types=scratch,
  )
  def k(*refs):
    args = refs[:len(data_args)]
    outs = refs[len(data_args):len(data_args) + n_out]
    tail = refs[len(data_args) + n_out:]
    didx, sidx0, sidx1, sidx2 = tail[:4]
    rows_slots = tail[4:4 + n_rows]
    rows0, rows1 = rows_slots[0], rows_slots[1]
    rows2 = rows_slots[2] if n_rows == 3 else None
    sg0, sg1, sg2, acc = tail[4 + n_rows:]
    z_h = args[z_i]
    c = lax.axis_index("c")
    s = lax.axis_index("s")

    def run_core(core, ph_idx, s_i, d_i):
      src2 = args[s_i] if s_i is not None else None
      dst1 = args[d_i]

      for p in range(n_phases):
        kind, _ = fetch_list[core][p]
        src_h = args[ph_idx[p]]
        out_h = outs[core * n_phases + p]

        pltpu.sync_copy(z_h.at[pl.ds(s * wz_acc, wz_acc)],
                        acc.at[pl.ds(s * wz_acc, wz_acc)])
        plsc.subcore_barrier()

        if kind == "gather":
          nb_g = dst1.size // IW
          start, cnt = _subcore_range(s, nb_g)
          nslot = 3
          ntrip = cnt // nslot
          slots = ((sidx0, didx.at[0], rows0, sg0),
                   (sidx1, didx.at[1], rows1, sg1),
                   (sidx2, didx.at[2], rows2, sg2))

          def idxload2(b, sv, dv):
            base = (start + b) * IW
            pltpu.sync_copy(src2.at[pl.ds(base, IW)], sv)
            pltpu.sync_copy(dst1.at[pl.ds(base, IW)], dv)

          def gwait(sv, rows, sem):
            pltpu.make_async_copy(src_h.at[sv], rows, sem).wait()

          for t, (sv, dv, rw, sem) in enumerate(slots):
            @pl.when(t < cnt)
            def _(t=t, sv=sv, dv=dv, rw=rw, sem=sem):
              idxload2(t, sv, dv)
              pltpu.async_copy(src_h.at[sv], rw, sem)

          def body(j, carry):
            for t, (sv, dv, rw, sem) in enumerate(slots):
              b = nslot * j + t
              gwait(sv, rw, sem)
              pltpu.sync_copy(rw, acc.at[dv], add=True)

              @pl.when(b + nslot < cnt)
              def _(b=b, sv=sv, dv=dv, rw=rw, sem=sem):
                idxload2(b + nslot, sv, dv)
                pltpu.async_copy(src_h.at[sv], rw, sem)
            return carry

          lax.fori_loop(0, ntrip, body, 0)

          rem = cnt - nslot * ntrip
          for t, (sv, dv, rw, sem) in enumerate(slots[:nslot - 1]):
            @pl.when(t < rem)
            def _(sv=sv, dv=dv, rw=rw, sem=sem):
              gwait(sv, rw, sem)
              pltpu.sync_copy(rw, acc.at[dv], add=True)
        elif kind == "lin16":
          nb_g = dst1.size // IW
          start, cnt = _subcore_range(s, nb_g)
          npairs = cnt // 2
          nv0, nv1, w = rows_slots
          d0, d1 = didx.at[0], didx.at[1]
          iota16 = lax.iota(jnp.int32, 16)
          e0 = jnp.where(iota16 == 0, 1.0, 0.0)
          z16 = jnp.zeros((16,), jnp.float32)
          for i in range(IW):
            w[i, pl.ds(16, 16)] = e0
            for kk in range(6):
              w[i, pl.ds(32 + 16 * kk, 16)] = z16

          def ld(b, nv, dv, sem):
            base = (start + b) * IW
            pltpu.async_copy(src_h.at[pl.ds(base, IW)], nv, sem)
            pltpu.sync_copy(dst1.at[pl.ds(base, IW)], dv)

          def ldwait(nv, sem):
            pltpu.make_async_copy(src_h.at[pl.ds(0, IW)], nv, sem).wait()

          def expand(nv):
            for i in range(IW):
              w[i, pl.ds(0, 16)] = nv[i, :]

          @pl.when(0 < cnt)
          def _():
            ld(0, nv0, d0, sg0)

          @pl.when(1 < cnt)
          def _():
            ld(1, nv1, d1, sg1)

          def body(j, carry):
            b0 = 2 * j
            b1 = b0 + 1
            ldwait(nv0, sg0)
            expand(nv0)
            pltpu.sync_copy(w, acc.at[d0], add=True)

            @pl.when(b0 + 2 < cnt)
            def _():
              ld(b0 + 2, nv0, d0, sg0)
            ldwait(nv1, sg1)
            expand(nv1)
            pltpu.sync_copy(w, acc.at[d1], add=True)

            @pl.when(b1 + 2 < cnt)
            def _():
              ld(b1 + 2, nv1, d1, sg1)
            return carry

          lax.fori_loop(0, npairs, body, 0)

          @pl.when(cnt - 2 * npairs == 1)
          def _():
            ldwait(nv0, sg0)
            expand(nv0)
            pltpu.sync_copy(w, acc.at[d0], add=True)

        plsc.subcore_barrier()
        _split_copy(acc, out_h, s, n_dst)
        plsc.subcore_barrier()

    for core in range(NC):
      ph_idx, s_i, d_i = layout[core]

      @pl.when(c == core)
      def _():
        run_core(core, ph_idx, s_i, d_i)

  return k(*data_args)


def _tc_epilogue(seg0, seg1, a, h0, h1, we_aug, wl, wr, b, g, bt, last):
  n = seg0.shape[0]
  blk = 2000
  grid = (n // blk,)

  def body(seg0_r, seg1_r, a_r, h0_r, h1_r, wea_r, wl_r, wr_r, b_r,
           g_r, bt_r, *outs):
    av = a_r[...]
    cnt = jnp.maximum(av[:, 16:17], 1.0)
    ea = jnp.dot(av, wea_r[...], preferred_element_type=jnp.float32,
                 precision=lax.Precision.HIGHEST)
    seg = jnp.concatenate([seg0_r[...], seg1_r[...]], axis=1) + ea
    agg = seg / cnt
    h = jnp.concatenate([h0_r[...], h1_r[...]], axis=1)
    y = (jnp.dot(agg, wl_r[...], preferred_element_type=jnp.float32,
                 precision=lax.Precision.HIGHEST)
         + jnp.dot(h, wr_r[...], preferred_element_type=jnp.float32,
                   precision=lax.Precision.HIGHEST)
         + b_r[...])
    mu = jnp.mean(y, axis=1, keepdims=True)
    var = jnp.mean((y - mu) ** 2, axis=1, keepdims=True)
    z = (y - mu) * lax.rsqrt(var + 1e-5) * g_r[...] + bt_r[...]
    if last:
      outs[0][...] = z
    else:
      outs[0][...] = z[:, :128]
      outs[1][...] = z[:, 128:]

  row_spec = lambda w: pl.BlockSpec((blk, w), lambda i: (i, 0))
  full_spec = lambda r, w: pl.BlockSpec((r, w), lambda i: (0, 0))
  in_specs = [row_spec(128), row_spec(128), row_spec(128), row_spec(128),
              row_spec(128), full_spec(128, 256), full_spec(256, 256),
              full_spec(256, 256), full_spec(1, 256), full_spec(1, 256),
              full_spec(1, 256)]
  if last:
    out_shape = jax.ShapeDtypeStruct((n, 256), jnp.float32)
    out_specs = row_spec(256)
  else:
    out_shape = (jax.ShapeDtypeStruct((n, 128), jnp.float32),) * 2
    out_specs = (row_spec(128), row_spec(128))

  return pl.pallas_call(
      body, grid=grid, in_specs=in_specs, out_specs=out_specs,
      out_shape=out_shape,
  )(seg0, seg1, a, h0, h1, we_aug, wl, wr, b.reshape(1, 256),
    g.reshape(1, 256), bt.reshape(1, 256))


def kernel(x_user, x_item, edge_index_user_buys_item,
           edge_index_item_rev_buys_user, edge_time_user_buys_item,
           edge_time_item_rev_buys_user, We, be, Wl, bl, Wr, br,
           gamma, beta):
  n_user, d = x_user.shape
  n_item = x_item.shape[0]
  e = edge_time_user_buys_item.shape[0]
  layers = Wl.shape[0]
  assert n_user == n_item and d == 256

  n_pad = ((n_item + NS * 8 - 1) // (NS * 8)) * (NS * 8)

  assert e % IW == 0
  src1_ui = edge_index_user_buys_item[0]
  dst1_ui = edge_index_user_buys_item[1]
  src1_iu = edge_index_item_rev_buys_user[0]
  dst1_iu = edge_index_item_rev_buys_user[1]

  zeros128 = jnp.zeros((n_pad, 128), jnp.float32)

  a_ui, a_iu = _sc_segsum(
      [[("lin16", edge_time_user_buys_item)],
       [("lin16", edge_time_item_rev_buys_user)]],
      [(None, dst1_ui), (None, dst1_iu)],
      zeros128, n_item, n_pad, 128)

  zpad = jnp.zeros((111, 256), jnp.float32)
  we_aug0 = jnp.concatenate([We[0], be[0][None, :], zpad], axis=0)
  we_aug1 = jnp.concatenate([We[1], be[1][None, :], zpad], axis=0)

  h_u0, h_u1 = x_user[:, :128], x_user[:, 128:]
  h_i0, h_i1 = x_item[:, :128], x_item[:, 128:]

  for l in range(layers):
    last = l == layers - 1
    seg_i0, seg_i1 = _sc_segsum(
        [[("gather", h_u0)], [("gather", h_u1)]],
        [(src1_ui, dst1_ui), (src1_ui, dst1_ui)],
        zeros128, n_item, n_pad, 128)
    out_i = _tc_epilogue(seg_i0, seg_i1, a_ui, h_i0, h_i1, we_aug0,
                         Wl[l, 0], Wr[l, 0], bl[l, 0] + br[l, 0],
                         gamma[1], beta[1], last)
    seg_u0, seg_u1 = _sc_segsum(
        [[("gather", h_i0)], [("gather", h_i1)]],
        [(src1_iu, dst1_iu), (src1_iu, dst1_iu)],
        zeros128, n_item, n_pad, 128)
    out_u = _tc_epilogue(seg_u0, seg_u1, a_iu, h_u0, h_u1, we_aug1,
                         Wl[l, 1], Wr[l, 1], bl[l, 1] + br[l, 1],
                         gamma[0], beta[0], last)
    if last:
      return out_u, out_i
    h_i0, h_i1 = out_i
    h_u0, h_u1 = out_u
